# baseline (speedup 1.0000x reference)
import functools

import jax
import jax.numpy as jnp
from jax import lax
from jax.experimental import pallas as pl
from jax.experimental.pallas import tpu as pltpu
from jax.experimental.pallas import tpu_sc as plsc

B = 16384
D = 64
NC = 2
NS = 16
L = 16
NW = NC * NS
BPW = B // NW
CHUNK = 128
NCHUNK = BPW // CHUNK
GROUPS = BPW // L


def _body(uid_hbm, iid_hbm, uemb_hbm, iemb_hbm, ubw_hbm, ibw_hbm, gb_hbm,
          out_hbm,
          uid_v, iid_v, urows_v, irows_v, ub_v, ib_v, gb_v, out_v,
          sem_u, sem_i, sem_ub, sem_ib):
    wid = lax.axis_index("s") * NC + lax.axis_index("c")
    base = wid * BPW

    for c in range(NCHUNK):
        src = pl.ds(base + c * CHUNK, CHUNK)
        pltpu.sync_copy(uid_hbm.at[src], uid_v.at[c])
        pltpu.sync_copy(iid_hbm.at[src], iid_v.at[c])
    pltpu.sync_copy(gb_hbm, gb_v.at[pl.ds(0, 1)])

    copies = []
    for c in range(NCHUNK):
        rsl = pl.ds(c * CHUNK, CHUNK)
        copies.append(pltpu.async_copy(uemb_hbm.at[uid_v.at[c]],
                                       urows_v.at[rsl], sem_u))
        copies.append(pltpu.async_copy(iemb_hbm.at[iid_v.at[c]],
                                       irows_v.at[rsl], sem_i))
        copies.append(pltpu.async_copy(ubw_hbm.at[uid_v.at[c]],
                                       ub_v.at[rsl], sem_ub))
        copies.append(pltpu.async_copy(ibw_hbm.at[iid_v.at[c]],
                                       ib_v.at[rsl], sem_ib))
    for cp in copies:
        cp.wait()

    gb = gb_v[pl.ds(0, L)][0]
    iota16 = lax.iota(jnp.int32, L)

    def group(g, carry):
        rows = g * L + iota16
        acc = plsc.load_gather(ub_v, [rows])
        acc = acc + plsc.load_gather(ib_v, [rows])
        acc = acc + gb
        for d in range(D):
            dd = jnp.full((L,), d, jnp.int32)
            u = plsc.load_gather(urows_v, [rows, dd])
            i = plsc.load_gather(irows_v, [rows, dd])
            acc = acc + u * i
        out_v[pl.ds(g * L, L)] = acc
        return carry

    lax.fori_loop(0, GROUPS, group, 0)

    pltpu.sync_copy(out_v, out_hbm.at[pl.ds(base, BPW)])


@jax.jit
def _mf_predict(user_ids, item_ids, user_emb, item_emb,
                user_bias_w, item_bias_w, global_bias):
    mesh = plsc.VectorSubcoreMesh(core_axis_name="c", subcore_axis_name="s",
                                  num_cores=NC, num_subcores=NS)
    kfn = pl.kernel(
        _body,
        out_type=jax.ShapeDtypeStruct((B,), jnp.float32),
        mesh=mesh,
        scratch_types=[
            pltpu.VMEM((NCHUNK, CHUNK), jnp.int32),
            pltpu.VMEM((NCHUNK, CHUNK), jnp.int32),
            pltpu.VMEM((BPW, D), jnp.float32),
            pltpu.VMEM((BPW, D), jnp.float32),
            pltpu.VMEM((BPW,), jnp.float32),
            pltpu.VMEM((BPW,), jnp.float32),
            pltpu.VMEM((L,), jnp.float32),
            pltpu.VMEM((BPW,), jnp.float32),
            pltpu.SemaphoreType.DMA,
            pltpu.SemaphoreType.DMA,
            pltpu.SemaphoreType.DMA,
            pltpu.SemaphoreType.DMA,
        ],
        compiler_params=pltpu.CompilerParams(needs_layout_passes=False,
                                             use_tc_tiling_on_sc=False),
    )
    return kfn(user_ids, item_ids, user_emb, item_emb,
               user_bias_w.reshape(-1), item_bias_w.reshape(-1), global_bias)


def kernel(user_ids, item_ids, user_emb, item_emb, user_bias_w, item_bias_w,
           global_bias):
    return _mf_predict(user_ids.astype(jnp.int32), item_ids.astype(jnp.int32),
                       user_emb, item_emb, user_bias_w, item_bias_w,
                       global_bias)
